# baseline (speedup 1.0000x reference)
import functools

import jax
import jax.numpy as jnp
from jax import lax
from jax.experimental import pallas as pl
from jax.experimental.pallas import tpu as pltpu
from jax.experimental.pallas import tpu_sc as plsc

_NC = 2
_NS = 16
_LANES = 16
_NBUF = 4


def _make_sc_kernel(nl_sc, rows_layer, nembed, batch, npeople):
    nw = _NC * _NS
    gpw = batch // nw
    rpwl = gpw * npeople
    ch = min(128, rpwl)
    nchunks = rpwl // ch
    gpc = ch // npeople
    nk = nembed // _LANES
    nt = nl_sc * nchunks
    nbuf = min(_NBUF, nt)
    assert nt % nbuf == 0

    mesh = plsc.VectorSubcoreMesh(core_axis_name="c", subcore_axis_name="s")

    @functools.partial(
        pl.kernel,
        out_type=jax.ShapeDtypeStruct((nl_sc * rows_layer, nembed),
                                      jnp.float32),
        mesh=mesh,
        scratch_types=[
            pltpu.VMEM((gpw,), jnp.int32),
            pltpu.VMEM((gpw, nembed), jnp.float32),
            [pltpu.VMEM((ch, nembed), jnp.float32)] * nbuf,
            pltpu.SemaphoreType.DMA,
            [pltpu.SemaphoreType.DMA] * nbuf,
            [pltpu.SemaphoreType.DMA] * nbuf,
        ],
    )
    def body(r_hbm, off_hbm, oe_hbm, out_hbm, offs_v, oerows_v,
             bufs, gsem, isems, osems):
        wid = lax.axis_index("s") * _NC + lax.axis_index("c")
        g0 = wid * gpw
        row0 = g0 * npeople

        def base_of(t):
            return (t // nchunks) * rows_layer + row0 + (t % nchunks) * ch

        def in_cp(t, b):
            return pltpu.make_async_copy(r_hbm.at[pl.ds(base_of(t), ch)],
                                         bufs[b], isems[b])

        def out_cp(t, b):
            return pltpu.make_async_copy(bufs[b],
                                         out_hbm.at[pl.ds(base_of(t), ch)],
                                         osems[b])

        for b in range(nbuf):
            in_cp(b, b).start()
        pltpu.sync_copy(off_hbm.at[pl.ds(g0, gpw)], offs_v)
        pltpu.async_copy(oe_hbm.at[offs_v], oerows_v, gsem).wait()

        def add_chunk(buf, c):
            def group_body(g, carry):
                addends = [oerows_v[c * gpc + g, pl.ds(k * _LANES, _LANES)]
                           for k in range(nk)]

                def row_body(i, carry2):
                    rr = g * npeople + i
                    for k in range(nk):
                        sl = pl.ds(k * _LANES, _LANES)
                        buf[rr, sl] = buf[rr, sl] + addends[k]
                    return carry2

                return lax.fori_loop(0, npeople, row_body, carry,
                                     unroll=4)

            lax.fori_loop(0, gpc, group_body, 0)

        def round_body(g, carry):
            for j in range(nbuf):
                u = g * nbuf + j
                b = j
                pb = (j - 1) % nbuf
                refill = jnp.logical_and(u >= 1, u + nbuf - 1 < nt)

                @pl.when(refill)
                def _():
                    out_cp(u - 1, pb).wait()
                    in_cp(u - 1 + nbuf, pb).start()

                in_cp(u, b).wait()
                add_chunk(bufs[b], (u % nchunks) if nchunks > 1 else 0)
                out_cp(u, b).start()
            return carry

        lax.fori_loop(0, nt // nbuf, round_body, 0)
        for t in range(nt - nbuf, nt):
            out_cp(t, t % nbuf).wait()

    return body


def _tc_body(off_ref, r_ref, oe_ref, out_ref):
    g_blk, _ = off_ref.shape
    max_len = oe_ref.shape[0]
    span = lax.broadcasted_iota(jnp.int32, (g_blk, max_len), 1)
    onehot = (off_ref[...] == span).astype(jnp.float32)
    rows = jax.lax.dot_general(
        onehot, oe_ref[...],
        dimension_numbers=(((1,), (0,)), ((), ())),
        preferred_element_type=jnp.float32)
    out_ref[...] = r_ref[...] + rows[None, :, None, :]


def _tc_add(r4, off2d, oe, l0, g_blk=128):
    nlayers, batch, npeople, nembed = r4.shape
    nl_tc = nlayers - l0
    grid = (nl_tc, batch // g_blk)
    return pl.pallas_call(
        _tc_body,
        grid=grid,
        in_specs=[
            pl.BlockSpec((g_blk, 1), lambda l, i: (i, 0)),
            pl.BlockSpec((1, g_blk, npeople, nembed),
                         lambda l, i: (l + l0, i, 0, 0)),
            pl.BlockSpec(oe.shape, lambda l, i: (0, 0)),
        ],
        out_specs=pl.BlockSpec((1, g_blk, npeople, nembed),
                               lambda l, i: (l, i, 0, 0)),
        out_shape=jax.ShapeDtypeStruct((nl_tc, batch, npeople, nembed),
                                       jnp.float32),
    )(off2d, r4, oe)


def kernel(r, offset, npeople, oe):
    nlayers, rows_layer, nembed = r.shape
    batch = offset.shape[0]
    np_static = rows_layer // batch
    nl_sc = nlayers // 2

    r2 = r.reshape(nlayers * rows_layer, nembed)
    sc_fn = _make_sc_kernel(nl_sc, rows_layer, nembed, batch, np_static)
    sc_out = sc_fn(r2, offset, oe)

    r4 = r.reshape(nlayers, batch, np_static, nembed)
    tc_out = _tc_add(r4, offset.reshape(batch, 1), oe, nl_sc)

    out = jnp.concatenate(
        [sc_out.reshape(nl_sc, rows_layer, nembed),
         tc_out.reshape(nlayers - nl_sc, rows_layer, nembed)], axis=0)
    return out


# speedup vs baseline: 1.4915x; 1.4915x over previous
import functools

import jax
import jax.numpy as jnp
from jax import lax
from jax.experimental import pallas as pl
from jax.experimental.pallas import tpu as pltpu
from jax.experimental.pallas import tpu_sc as plsc

_NC = 2
_NS = 16
_LANES = 16
_NBUF = 8
_CHUNK = 64


def _make_sc_kernel(nlayers, rows_layer, nembed, batch, npeople):
    nw = _NC * _NS
    gpw = batch // nw
    rpwl = gpw * npeople
    ch = min(_CHUNK, rpwl)
    nchunks = rpwl // ch
    gpc = ch // npeople
    nk = nembed // _LANES
    nt = nlayers * nchunks
    nbuf = min(_NBUF, nt)
    assert nt % nbuf == 0

    mesh = plsc.VectorSubcoreMesh(core_axis_name="c", subcore_axis_name="s")

    @functools.partial(
        pl.kernel,
        out_type=jax.ShapeDtypeStruct((nlayers * rows_layer, nembed),
                                      jnp.float32),
        mesh=mesh,
        scratch_types=[
            pltpu.VMEM((gpw,), jnp.int32),
            pltpu.VMEM((gpw, nembed), jnp.float32),
            [pltpu.VMEM((ch, nembed), jnp.float32)] * nbuf,
            pltpu.SemaphoreType.DMA,
            [pltpu.SemaphoreType.DMA] * nbuf,
            [pltpu.SemaphoreType.DMA] * nbuf,
        ],
    )
    def body(r_hbm, off_hbm, oe_hbm, out_hbm, offs_v, oerows_v,
             bufs, gsem, isems, osems):
        wid = lax.axis_index("s") * _NC + lax.axis_index("c")
        g0 = wid * gpw
        row0 = g0 * npeople

        def base_of(t):
            return (t // nchunks) * rows_layer + row0 + (t % nchunks) * ch

        def in_cp(t, b):
            return pltpu.make_async_copy(r_hbm.at[pl.ds(base_of(t), ch)],
                                         bufs[b], isems[b])

        def out_cp(t, b):
            return pltpu.make_async_copy(bufs[b],
                                         out_hbm.at[pl.ds(base_of(t), ch)],
                                         osems[b])

        for b in range(nbuf):
            in_cp(b, b).start()
        pltpu.sync_copy(off_hbm.at[pl.ds(g0, gpw)], offs_v)
        pltpu.async_copy(oe_hbm.at[offs_v], oerows_v, gsem).wait()

        def add_chunk(buf, c):
            def group_body(g, carry):
                addends = [oerows_v[c * gpc + g, pl.ds(k * _LANES, _LANES)]
                           for k in range(nk)]

                def row_body(i, carry2):
                    rr = g * npeople + i
                    for k in range(nk):
                        sl = pl.ds(k * _LANES, _LANES)
                        buf[rr, sl] = buf[rr, sl] + addends[k]
                    return carry2

                return lax.fori_loop(0, npeople, row_body, carry,
                                     unroll=4)

            lax.fori_loop(0, gpc, group_body, 0)

        def round_body(g, carry):
            for j in range(nbuf):
                u = g * nbuf + j
                b = j
                pb = (j - 1) % nbuf
                refill = jnp.logical_and(u >= 1, u + nbuf - 1 < nt)

                @pl.when(refill)
                def _():
                    out_cp(u - 1, pb).wait()
                    in_cp(u - 1 + nbuf, pb).start()

                in_cp(u, b).wait()
                add_chunk(bufs[b], (u % nchunks) if nchunks > 1 else 0)
                out_cp(u, b).start()
            return carry

        lax.fori_loop(0, nt // nbuf, round_body, 0)
        for t in range(nt - nbuf, nt):
            out_cp(t, t % nbuf).wait()

    return body


def kernel(r, offset, npeople, oe):
    nlayers, rows_layer, nembed = r.shape
    batch = offset.shape[0]
    np_static = rows_layer // batch
    r2 = r.reshape(nlayers * rows_layer, nembed)
    fn = _make_sc_kernel(nlayers, rows_layer, nembed, batch, np_static)
    out = fn(r2, offset, oe)
    return out.reshape(nlayers, rows_layer, nembed)


# speedup vs baseline: 1.4972x; 1.0038x over previous
import functools

import jax
import jax.numpy as jnp
from jax import lax
from jax.experimental import pallas as pl
from jax.experimental.pallas import tpu as pltpu
from jax.experimental.pallas import tpu_sc as plsc

_NC = 2
_NS = 16
_LANES = 16
_NBUF = 4
_CHUNK = 128


def _make_sc_kernel(nlayers, rows_layer, nembed, batch, npeople):
    nw = _NC * _NS
    gpw = batch // nw
    rpwl = gpw * npeople
    ch = min(_CHUNK, rpwl)
    nchunks = rpwl // ch
    gpc = ch // npeople
    nk = nembed // _LANES
    nt = nlayers * nchunks
    nbuf = min(_NBUF, nt)
    assert nt % nbuf == 0

    mesh = plsc.VectorSubcoreMesh(core_axis_name="c", subcore_axis_name="s")

    @functools.partial(
        pl.kernel,
        out_type=jax.ShapeDtypeStruct((nlayers * rows_layer, nembed),
                                      jnp.float32),
        mesh=mesh,
        scratch_types=[
            pltpu.VMEM((gpw,), jnp.int32),
            pltpu.VMEM((gpw, nembed), jnp.float32),
            [pltpu.VMEM((ch, nembed), jnp.float32)] * nbuf,
            pltpu.SemaphoreType.DMA,
            [pltpu.SemaphoreType.DMA] * nbuf,
            [pltpu.SemaphoreType.DMA] * nbuf,
        ],
    )
    def body(r_hbm, off_hbm, oe_hbm, out_hbm, offs_v, oerows_v,
             bufs, gsem, isems, osems):
        wid = lax.axis_index("s") * _NC + lax.axis_index("c")
        g0 = wid * gpw
        row0 = g0 * npeople

        def base_of(t):
            return (t // nchunks) * rows_layer + row0 + (t % nchunks) * ch

        def in_cp(t, b):
            return pltpu.make_async_copy(r_hbm.at[pl.ds(base_of(t), ch)],
                                         bufs[b], isems[b])

        def out_cp(t, b):
            return pltpu.make_async_copy(bufs[b],
                                         out_hbm.at[pl.ds(base_of(t), ch)],
                                         osems[b])

        for b in range(nbuf):
            in_cp(b, b).start()
        pltpu.sync_copy(off_hbm.at[pl.ds(g0, gpw)], offs_v)
        pltpu.async_copy(oe_hbm.at[offs_v], oerows_v, gsem).wait()

        def add_chunk(buf, c):
            def group_body(g, carry):
                addends = [oerows_v[c * gpc + g, pl.ds(k * _LANES, _LANES)]
                           for k in range(nk)]

                def row_body(i, carry2):
                    rr = g * npeople + i
                    for k in range(nk):
                        sl = pl.ds(k * _LANES, _LANES)
                        buf[rr, sl] = buf[rr, sl] + addends[k]
                    return carry2

                return lax.fori_loop(0, npeople, row_body, carry,
                                     unroll=4)

            lax.fori_loop(0, gpc, group_body, 0)

        def round_body(g, carry):
            for j in range(nbuf):
                u = g * nbuf + j
                b = j
                pb = (j - 1) % nbuf
                refill = jnp.logical_and(u >= 1, u + nbuf - 1 < nt)

                @pl.when(refill)
                def _():
                    out_cp(u - 1, pb).wait()
                    in_cp(u - 1 + nbuf, pb).start()

                in_cp(u, b).wait()
                add_chunk(bufs[b], (u % nchunks) if nchunks > 1 else 0)
                out_cp(u, b).start()
            return carry

        lax.fori_loop(0, nt // nbuf, round_body, 0)
        for t in range(nt - nbuf, nt):
            out_cp(t, t % nbuf).wait()

    return body


def kernel(r, offset, npeople, oe):
    nlayers, rows_layer, nembed = r.shape
    batch = offset.shape[0]
    np_static = rows_layer // batch
    r2 = r.reshape(nlayers * rows_layer, nembed)
    fn = _make_sc_kernel(nlayers, rows_layer, nembed, batch, np_static)
    out = fn(r2, offset, oe)
    return out.reshape(nlayers, rows_layer, nembed)
